# baseline (speedup 1.0000x reference)
import jax
import jax.numpy as jnp
from jax.experimental import pallas as pl
from jax.experimental.pallas import tpu as pltpu

_B, _S, _D = 4, 4096, 2048


def _pool_body(mask_hbm, hs_ref, out_ref, mask_v, msem, sem):
    mcp = pltpu.make_async_copy(mask_hbm, mask_v, msem)
    mcp.start()
    scp = pltpu.make_async_copy(hs_ref.at[:, _S - 1, :], out_ref, sem)
    scp.start()
    mcp.wait()
    idxs = [jnp.sum(mask_v[b, :]) - 1 for b in range(_B)]
    scp.wait()
    for b in range(_B):
        idx = idxs[b]

        @pl.when(idx != _S - 1)
        def _fix(b=b, idx=idx):
            cp = pltpu.make_async_copy(
                hs_ref.at[b, pl.ds(idx, 1), :], out_ref.at[pl.ds(b, 1), :], sem)
            cp.start()
            cp.wait()


def kernel(last_hidden_state, attention_mask):
    mask = attention_mask.astype(jnp.int32)
    return pl.pallas_call(
        _pool_body,
        out_shape=jax.ShapeDtypeStruct((_B, _D), jnp.float32),
        in_specs=[
            pl.BlockSpec(memory_space=pl.ANY),
            pl.BlockSpec(memory_space=pl.ANY),
        ],
        out_specs=pl.BlockSpec(memory_space=pl.ANY),
        scratch_shapes=[
            pltpu.VMEM((_B, _S), jnp.int32),
            pltpu.SemaphoreType.DMA,
            pltpu.SemaphoreType.DMA,
        ],
    )(mask, last_hidden_state)
